# baseline (speedup 1.0000x reference)
import jax
import jax.numpy as jnp
from jax.experimental import pallas as pl
from jax.experimental.pallas import tpu as pltpu
from jax.experimental.pallas import tpu_sc as plsc

_VOCAB = 1000000
_HALF = _VOCAB // 2
_EMB = 64
_HID = 128
_B = 4096
_L = 50
_N = _B * _L

_GATHER_WINDOW = 128
_REPACK_BLK = 5000
_BB = 16


def _tc_repack(table):
    def body(lo_ref, hi_ref, o_ref):
        o_ref[:, :_EMB] = lo_ref[...]
        o_ref[:, _EMB:] = hi_ref[...]

    nblk = _HALF // _REPACK_BLK
    return pl.pallas_call(
        body,
        grid=(nblk,),
        in_specs=[
            pl.BlockSpec((_REPACK_BLK, _EMB), lambda i: (i, 0)),
            pl.BlockSpec((_REPACK_BLK, _EMB), lambda i: (i + nblk, 0)),
        ],
        out_specs=pl.BlockSpec((_REPACK_BLK, 2 * _EMB), lambda i: (i, 0)),
        out_shape=jax.ShapeDtypeStruct((_HALF, 2 * _EMB), jnp.float32),
    )(table, table)


def _sc_gather(table2, idx_flat):
    mesh = plsc.VectorSubcoreMesh(core_axis_name="core", subcore_axis_name="subcore")

    @pl.kernel(
        out_type=jax.ShapeDtypeStruct((_N, 2 * _EMB), table2.dtype),
        mesh=mesh,
    )
    def gather_kernel(tab_hbm, i_hbm, o_hbm):
        def body(i_vmem, o_vmem):
            pltpu.sync_copy(tab_hbm.at[i_vmem.at[0]], o_vmem)

        pltpu.emit_pipeline(
            body,
            grid=(_N // _GATHER_WINDOW,),
            in_specs=[pl.BlockSpec((1, _GATHER_WINDOW), index_map=lambda i: (0, i))],
            out_specs=[pl.BlockSpec((_GATHER_WINDOW, 2 * _EMB), index_map=lambda i: (i, 0))],
            core_axis_name=("core", "subcore"),
            dimension_semantics=(pltpu.PARALLEL,),
        )(i_hbm, o_hbm)

    return gather_kernel(table2, idx_flat)


def _tc_dense(g, selb, W, b2d):
    def body(g_ref, s_ref, w_ref, b_ref, o_ref):
        gv = g_ref[...]
        sel = s_ref[...] != 0
        e = jnp.where(sel[:, :_EMB], gv[:, _EMB:], gv[:, :_EMB])
        h = jnp.tanh(e)
        acc = jnp.dot(h, w_ref[...], preferred_element_type=jnp.float32,
                      precision=jax.lax.Precision.HIGHEST)
        hv = jnp.tanh(acc + b_ref[...])
        for j in range(_BB):
            o_ref[j] = hv[j * _L:(j + 1) * _L, :]

    rows = _BB * _L
    return pl.pallas_call(
        body,
        grid=(_B // _BB,),
        in_specs=[
            pl.BlockSpec((rows, 2 * _EMB), lambda i: (i, 0)),
            pl.BlockSpec((rows, _HID), lambda i: (i, 0)),
            pl.BlockSpec((_EMB, _HID), lambda i: (0, 0)),
            pl.BlockSpec((1, _HID), lambda i: (0, 0)),
        ],
        out_specs=pl.BlockSpec((_BB, _L, _HID), lambda i: (i, 0, 0)),
        out_shape=jax.ShapeDtypeStruct((_B, _L, _HID), jnp.float32),
    )(g, selb, W, b2d)


def kernel(x, table, W, b):
    xf = x.reshape(_N)
    idx2 = jnp.where(xf < _HALF, xf, xf - _HALF).reshape(1, _N)
    selb = jnp.broadcast_to(
        (xf >= _HALF).astype(jnp.int8).reshape(_N, 1), (_N, _HID))
    table2 = _tc_repack(table)
    g = _sc_gather(table2, idx2)
    return _tc_dense(g, selb, W, b.reshape(1, _HID))
